# Initial kernel scaffold; baseline (speedup 1.0000x reference)
import functools

import jax
import jax.numpy as jnp
from jax import lax
from jax.experimental import pallas as pl
from jax.experimental.pallas import tpu as pltpu
from jax.experimental.pallas import tpu_sc as plsc

VOCAB = 100000
HID = 128
B = 4096
L = 50

NC = 2
NS = 16
NW = NC * NS
NTOT = B * L
C = 128
NCH = NTOT // C
CH_W = NCH // NW
ROWS_W = NTOT // NW

_mesh = plsc.VectorSubcoreMesh(core_axis_name="c", subcore_axis_name="s")


@functools.partial(
    pl.kernel,
    out_type=jax.ShapeDtypeStruct((NTOT, HID), jnp.float32),
    mesh=_mesh,
    scratch_types=[
        pltpu.VMEM((CH_W, C), jnp.int32),
        pltpu.VMEM((C, HID), jnp.float32),
        pltpu.SemaphoreType.DMA,
    ],
)
def _embed_gather(idx_hbm, table_hbm, out_hbm, idx_v, rows, gsem):
    wid = lax.axis_index("s") * NC + lax.axis_index("c")
    base_ch = wid * CH_W
    base_row = wid * ROWS_W
    pltpu.sync_copy(idx_hbm.at[pl.ds(base_ch, CH_W)], idx_v)

    def step(j, carry):
        pltpu.async_copy(table_hbm.at[idx_v.at[j]], rows, gsem).wait()
        off = pl.multiple_of(base_row + j * C, C)
        pltpu.sync_copy(rows, out_hbm.at[pl.ds(off, C)])
        return carry

    lax.fori_loop(0, CH_W, step, 0)


def kernel(source, table):
    idx = source.reshape(NCH, C).astype(jnp.int32)
    out = _embed_gather(idx, table)
    return out.reshape(B, L, HID)


# speedup vs baseline: 3.0776x; 3.0776x over previous
import functools

import jax
import jax.numpy as jnp
from jax import lax
from jax.experimental import pallas as pl
from jax.experimental.pallas import tpu as pltpu
from jax.experimental.pallas import tpu_sc as plsc

VOCAB = 100000
HID = 128
B = 4096
L = 50

NC = 2
NS = 16
NW = NC * NS
NTOT = B * L
C = 128
NCH = NTOT // C
CH_W = NCH // NW
ROWS_W = NTOT // NW

_mesh = plsc.VectorSubcoreMesh(core_axis_name="c", subcore_axis_name="s")


@functools.partial(
    pl.kernel,
    out_type=jax.ShapeDtypeStruct((NTOT, HID), jnp.float32),
    mesh=_mesh,
    scratch_types=[
        pltpu.VMEM((CH_W, C), jnp.int32),
        pltpu.VMEM((C, HID), jnp.float32),
        pltpu.SemaphoreType.DMA,
    ],
)
def _embed_gather(idx_hbm, table_hbm, out_hbm, idx_v, rows, gsem):
    wid = lax.axis_index("s") * NC + lax.axis_index("c")
    base_row = wid * ROWS_W
    pltpu.sync_copy(idx_hbm.at[wid], idx_v)

    def step(j, carry):
        pltpu.async_copy(table_hbm.at[idx_v.at[j]], rows, gsem).wait()
        off = pl.multiple_of(base_row + j * C, C)
        pltpu.sync_copy(rows, out_hbm.at[pl.ds(off, C)])
        return carry

    lax.fori_loop(0, CH_W, step, 0)


def kernel(source, table):
    idx = source.reshape(NW, CH_W, C).astype(jnp.int32)
    out = _embed_gather(idx, table)
    return out.reshape(B, L, HID)
